# Initial kernel scaffold; baseline (speedup 1.0000x reference)
import jax
import jax.numpy as jnp
from jax.experimental import pallas as pl


def kernel(x, hyperedge_index, W0v, b0v, W0e, b0e, W1v, b1v, W1e, b1e):
    raise NotImplementedError("write your pallas kernel here")


# speedup vs baseline: 3.6256x; 3.6256x over previous
import functools

import jax
import jax.numpy as jnp
from jax import lax
from jax.experimental import pallas as pl
from jax.experimental.pallas import tpu as pltpu
from jax.experimental.pallas import tpu_sc as plsc

N = 10000
D = 128
NNZ = 320000
R = 10240
C = 128
NC = 2
NS = 16
NW = NC * NS
CPT = 80
NNZ_PAD = NW * CPT * C
PAD_IDX = 10000
RPT = R // NS


def _wid(c, s):
    return c * NS + s


def _sc_mesh():
    return plsc.VectorSubcoreMesh(core_axis_name="c", subcore_axis_name="s",
                                  num_cores=NC, num_subcores=NS)


def _spmm_body(tab_hbm, gidx_hbm, sidx_hbm, zrows_hbm, out_hbm,
               gbuf, sbuf, rows, acc):
    c = lax.axis_index("c")
    s = lax.axis_index("s")
    base = _wid(c, s) * CPT

    pltpu.sync_copy(zrows_hbm, acc.at[pl.ds(s * RPT, RPT)])
    plsc.subcore_barrier()

    def chunk(j, carry):
        off = pl.multiple_of((base + j) * C, C)
        pltpu.sync_copy(gidx_hbm.at[pl.ds(off, C)], gbuf.at[0])
        pltpu.sync_copy(sidx_hbm.at[pl.ds(off, C)], sbuf.at[0])
        pltpu.sync_copy(tab_hbm.at[gbuf.at[0]], rows.at[0])
        pltpu.sync_copy(rows.at[0], acc.at[sbuf.at[0]], add=True)
        return carry

    lax.fori_loop(0, CPT, chunk, 0)
    plsc.subcore_barrier()

    row0 = c * R + s * RPT
    pltpu.sync_copy(acc.at[pl.ds(s * RPT, RPT)], out_hbm.at[pl.ds(row0, RPT)])


@jax.jit
def _spmm(tab, gidx, sidx, zrows):
    return pl.kernel(
        _spmm_body,
        out_type=jax.ShapeDtypeStruct((NC * R, D), jnp.float32),
        mesh=_sc_mesh(),
        scratch_types=[
            pltpu.VMEM((1, C), jnp.int32),
            pltpu.VMEM((1, C), jnp.int32),
            pltpu.VMEM((1, C, D), jnp.float32),
            pltpu.VMEM_SHARED((R, D), jnp.float32),
        ],
    )(tab, gidx, sidx, zrows)


def _degrees_body(src_hbm, eid_hbm, zvec_hbm, out_hbm,
                  ibuf, ones, dv_acc, de_acc):
    c = lax.axis_index("c")
    s = lax.axis_index("s")
    base = _wid(c, s) * CPT

    for k in range(C // 16):
        ones[pl.ds(k * 16, 16)] = jnp.ones((16,), jnp.float32)
    pltpu.sync_copy(zvec_hbm, dv_acc.at[pl.ds(s * RPT, RPT)])
    pltpu.sync_copy(zvec_hbm, de_acc.at[pl.ds(s * RPT, RPT)])
    plsc.subcore_barrier()

    def chunk(j, carry):
        off = pl.multiple_of((base + j) * C, C)
        pltpu.sync_copy(src_hbm.at[pl.ds(off, C)], ibuf.at[0])
        pltpu.sync_copy(ones, dv_acc.at[ibuf.at[0]], add=True)
        pltpu.sync_copy(eid_hbm.at[pl.ds(off, C)], ibuf.at[0])
        pltpu.sync_copy(ones, de_acc.at[ibuf.at[0]], add=True)
        return carry

    lax.fori_loop(0, CPT, chunk, 0)
    plsc.subcore_barrier()

    pltpu.sync_copy(dv_acc.at[pl.ds(s * RPT, RPT)],
                    out_hbm.at[pl.ds((c * 2 + 0) * R + s * RPT, RPT)])
    pltpu.sync_copy(de_acc.at[pl.ds(s * RPT, RPT)],
                    out_hbm.at[pl.ds((c * 2 + 1) * R + s * RPT, RPT)])


@jax.jit
def _degrees(src, eid, zvec):
    return pl.kernel(
        _degrees_body,
        out_type=jax.ShapeDtypeStruct((NC * 2 * R,), jnp.float32),
        mesh=_sc_mesh(),
        scratch_types=[
            pltpu.VMEM((1, C), jnp.int32),
            pltpu.VMEM((C,), jnp.float32),
            pltpu.VMEM_SHARED((R,), jnp.float32),
            pltpu.VMEM_SHARED((R,), jnp.float32),
        ],
    )(src, eid, zvec)


def _norms_body(src_hbm, eid_hbm, dvb_hbm, dea_hbm, zvec_hbm, out_hbm,
                sbuf, ebuf, vals, en_acc, vn_acc):
    c = lax.axis_index("c")
    s = lax.axis_index("s")
    base = _wid(c, s) * CPT

    pltpu.sync_copy(zvec_hbm, en_acc.at[pl.ds(s * RPT, RPT)])
    pltpu.sync_copy(zvec_hbm, vn_acc.at[pl.ds(s * RPT, RPT)])
    plsc.subcore_barrier()

    def chunk(j, carry):
        off = pl.multiple_of((base + j) * C, C)
        pltpu.sync_copy(src_hbm.at[pl.ds(off, C)], sbuf.at[0])
        pltpu.sync_copy(eid_hbm.at[pl.ds(off, C)], ebuf.at[0])
        pltpu.sync_copy(dvb_hbm.at[sbuf.at[0]], vals.at[0])
        pltpu.sync_copy(vals.at[0], en_acc.at[ebuf.at[0]], add=True)
        pltpu.sync_copy(dea_hbm.at[ebuf.at[0]], vals.at[0])
        pltpu.sync_copy(vals.at[0], vn_acc.at[sbuf.at[0]], add=True)
        return carry

    lax.fori_loop(0, CPT, chunk, 0)
    plsc.subcore_barrier()

    pltpu.sync_copy(en_acc.at[pl.ds(s * RPT, RPT)],
                    out_hbm.at[pl.ds((c * 2 + 0) * R + s * RPT, RPT)])
    pltpu.sync_copy(vn_acc.at[pl.ds(s * RPT, RPT)],
                    out_hbm.at[pl.ds((c * 2 + 1) * R + s * RPT, RPT)])


@jax.jit
def _norms(src, eid, dv_beta, de_alpha, zvec):
    return pl.kernel(
        _norms_body,
        out_type=jax.ShapeDtypeStruct((NC * 2 * R,), jnp.float32),
        mesh=_sc_mesh(),
        scratch_types=[
            pltpu.VMEM((1, C), jnp.int32),
            pltpu.VMEM((1, C), jnp.int32),
            pltpu.VMEM((1, C), jnp.float32),
            pltpu.VMEM_SHARED((R,), jnp.float32),
            pltpu.VMEM_SHARED((R,), jnp.float32),
        ],
    )(src, eid, dv_beta, de_alpha, zvec)


def _prep_tc_body(parts_ref, dvb_ref, dea_ref):
    dv = jnp.maximum(parts_ref[0, 0] + parts_ref[1, 0], 1.0)
    de = jnp.maximum(parts_ref[0, 1] + parts_ref[1, 1], 1.0)
    dvb_ref[...] = lax.rsqrt(dv)
    r = lax.rsqrt(de)
    dea_ref[...] = r * r * r


@jax.jit
def _prep_tc(deg_parts):
    return pl.pallas_call(
        _prep_tc_body,
        out_shape=(jax.ShapeDtypeStruct((R,), jnp.float32),
                   jax.ShapeDtypeStruct((R,), jnp.float32)),
    )(deg_parts)


def _norminv_tc_body(parts_ref, ei_ref, vi_ref):
    ei_ref[...] = 1.0 / jnp.maximum(parts_ref[0, 0] + parts_ref[1, 0], 1e-12)
    vi_ref[...] = 1.0 / jnp.maximum(parts_ref[0, 1] + parts_ref[1, 1], 1e-12)


@jax.jit
def _norminv_tc(norm_parts):
    return pl.pallas_call(
        _norminv_tc_body,
        out_shape=(jax.ShapeDtypeStruct((R,), jnp.float32),
                   jax.ShapeDtypeStruct((R,), jnp.float32)),
    )(norm_parts)


_BR = 1024


def _mm_body(x_ref, w_ref, b_ref, scale_ref, o_ref):
    y = jnp.dot(x_ref[...], w_ref[...], preferred_element_type=jnp.float32)
    o_ref[...] = (y + b_ref[...]) * scale_ref[...]


@jax.jit
def _mm_tc(x, w, b, scale):
    return pl.pallas_call(
        _mm_body,
        grid=(R // _BR,),
        in_specs=[
            pl.BlockSpec((_BR, D), lambda i: (i, 0)),
            pl.BlockSpec((D, D), lambda i: (0, 0)),
            pl.BlockSpec((1, D), lambda i: (0, 0)),
            pl.BlockSpec((_BR, 1), lambda i: (i, 0)),
        ],
        out_specs=pl.BlockSpec((_BR, D), lambda i: (i, 0)),
        out_shape=jax.ShapeDtypeStruct((R, D), jnp.float32),
    )(x, w, b, scale)


def _stage_body(parts_ref, inv_ref, w_ref, b_ref, scale_ref, o_ref):
    z = (parts_ref[0] + parts_ref[1]) * inv_ref[...]
    z = jnp.maximum(z, 0.0)
    y = jnp.dot(z, w_ref[...], preferred_element_type=jnp.float32)
    o_ref[...] = (y + b_ref[...]) * scale_ref[...]


@jax.jit
def _stage_tc(parts, inv, w, b, scale):
    return pl.pallas_call(
        _stage_body,
        grid=(R // _BR,),
        in_specs=[
            pl.BlockSpec((NC, _BR, D), lambda i: (0, i, 0)),
            pl.BlockSpec((_BR, 1), lambda i: (i, 0)),
            pl.BlockSpec((D, D), lambda i: (0, 0)),
            pl.BlockSpec((1, D), lambda i: (0, 0)),
            pl.BlockSpec((_BR, 1), lambda i: (i, 0)),
        ],
        out_specs=pl.BlockSpec((_BR, D), lambda i: (i, 0)),
        out_shape=jax.ShapeDtypeStruct((R, D), jnp.float32),
    )(parts, inv, w, b, scale)


def _combine_body(parts_ref, inv_ref, o_ref, *, relu):
    z = (parts_ref[0] + parts_ref[1]) * inv_ref[...]
    if relu:
        z = jnp.maximum(z, 0.0)
    o_ref[...] = z


@functools.partial(jax.jit, static_argnames=("relu",))
def _combine_tc(parts, inv, relu):
    return pl.pallas_call(
        functools.partial(_combine_body, relu=relu),
        grid=(R // _BR,),
        in_specs=[
            pl.BlockSpec((NC, _BR, D), lambda i: (0, i, 0)),
            pl.BlockSpec((_BR, 1), lambda i: (i, 0)),
        ],
        out_specs=pl.BlockSpec((_BR, D), lambda i: (i, 0)),
        out_shape=jax.ShapeDtypeStruct((R, D), jnp.float32),
    )(parts, inv)


def kernel(x, hyperedge_index, W0v, b0v, W0e, b0e, W1v, b1v, W1e, b1e):
    src = hyperedge_index[0]
    eid = hyperedge_index[1]
    pad = jnp.full((NNZ_PAD - NNZ,), PAD_IDX, jnp.int32)
    src_p = jnp.concatenate([src, pad])
    eid_p = jnp.concatenate([eid, pad])

    x_p = jnp.pad(x, ((0, R - N), (0, 0)))
    zrows = jnp.zeros((RPT, D), jnp.float32)
    zvec = jnp.zeros((RPT,), jnp.float32)

    deg_parts = _degrees(src_p, eid_p, zvec).reshape(NC, 2, R)
    dv_beta, de_alpha = _prep_tc(deg_parts)

    norm_parts = _norms(src_p, eid_p, dv_beta, de_alpha, zvec).reshape(NC, 2, R)
    e_inv, v_inv = _norminv_tc(norm_parts)

    dvb2 = dv_beta.reshape(R, 1)
    dea2 = de_alpha.reshape(R, 1)
    ei2 = e_inv.reshape(R, 1)
    vi2 = v_inv.reshape(R, 1)

    def one_layer(h, Wv, bv, We, be):
        hs = _mm_tc(h, Wv, bv.reshape(1, D), dvb2)
        e_parts = _spmm(hs, src_p, eid_p, zrows).reshape(NC, R, D)
        h2s = _stage_tc(e_parts, ei2, We, be.reshape(1, D), dea2)
        n_parts = _spmm(h2s, eid_p, src_p, zrows).reshape(NC, R, D)
        return n_parts

    n0 = one_layer(x_p, W0v, b0v, W0e, b0e)
    h1 = _combine_tc(n0, vi2, relu=True)
    n1 = one_layer(h1, W1v, b1v, W1e, b1e)
    out = _combine_tc(n1, vi2, relu=False)
    return out[:N]
